# Initial kernel scaffold; baseline (speedup 1.0000x reference)
import jax
import jax.numpy as jnp
from jax.experimental import pallas as pl


def kernel(true_batch_size, node_list, edge_index, fw_adj_matrix, bw_adj_matrix, W_sim, W_fw, W_bw, W_cls, b_cls):
    raise NotImplementedError("write your pallas kernel here")


# speedup vs baseline: 17.9977x; 17.9977x over previous
import functools

import jax
import jax.numpy as jnp
from jax import lax
from jax.experimental import pallas as pl
from jax.experimental.pallas import tpu as pltpu
from jax.experimental.pallas import tpu_sc as plsc

_EDGE_CHUNK = 128
_NUM_SC = 2
_NUM_TILES = 16


def _sc_scatter_partials(flat, eidx3):
    BN, F = flat.shape
    _, nchunks_total, CH = eidx3.shape
    NW = _NUM_SC * _NUM_TILES
    chunks_per_worker = nchunks_total // NW
    rows_per_tile = BN // _NUM_TILES

    mesh = plsc.VectorSubcoreMesh(core_axis_name="c", subcore_axis_name="s")

    @functools.partial(
        pl.kernel,
        out_type=jax.ShapeDtypeStruct((_NUM_SC, BN, F), jnp.float32),
        mesh=mesh,
        scratch_types=[
            pltpu.VMEM((chunks_per_worker, CH), jnp.int32),
            pltpu.VMEM((chunks_per_worker, CH), jnp.int32),
            pltpu.VMEM((CH, F), jnp.float32),
            pltpu.VMEM((CH, F), jnp.float32),
            pltpu.VMEM((CH, F), jnp.float32),
            pltpu.VMEM_SHARED((BN, F), jnp.float32),
            pltpu.SemaphoreType.DMA,
            pltpu.SemaphoreType.DMA,
        ],
        compiler_params=pltpu.CompilerParams(use_tc_tiling_on_sc=False),
    )
    def k(flat_hbm, eidx_hbm, out_hbm, src_v, dst_v, buf0, buf1, zbuf, acc,
          sem0, sem1):
        c = lax.axis_index("c")
        s = lax.axis_index("s")
        wid = c * _NUM_TILES + s

        @pl.loop(0, CH)
        def _(i):
            zbuf[i, :] = jnp.zeros((F,), jnp.float32)

        row0 = s * rows_per_tile

        @pl.loop(0, rows_per_tile // CH)
        def _(i):
            pltpu.sync_copy(zbuf, acc.at[pl.ds(row0 + i * CH, CH)])

        cbase = wid * chunks_per_worker
        pltpu.sync_copy(eidx_hbm.at[0, pl.ds(cbase, chunks_per_worker)], src_v)
        pltpu.sync_copy(eidx_hbm.at[1, pl.ds(cbase, chunks_per_worker)], dst_v)
        plsc.subcore_barrier()

        def start_gather(j, buf, sem):
            pltpu.async_copy(flat_hbm.at[src_v.at[j]], buf, sem)

        def wait_gather(j, buf, sem):
            pltpu.make_async_copy(flat_hbm.at[src_v.at[j]], buf, sem).wait()

        def scatter_add(j, buf):
            pltpu.sync_copy(buf, acc.at[dst_v.at[j]], add=True)

        start_gather(0, buf0, sem0)

        @pl.loop(0, chunks_per_worker, step=2)
        def _(j):
            wait_gather(j, buf0, sem0)
            start_gather(j + 1, buf1, sem1)
            scatter_add(j, buf0)
            wait_gather(j + 1, buf1, sem1)

            @pl.when(j + 2 < chunks_per_worker)
            def _():
                start_gather(j + 2, buf0, sem0)

            scatter_add(j + 1, buf1)

        plsc.subcore_barrier()
        pltpu.sync_copy(acc.at[pl.ds(row0, rows_per_tile)],
                        out_hbm.at[c, pl.ds(row0, rows_per_tile)])

    return k(flat, eidx3)


def _tc_dense(afw, abw, x, wf, wb):
    B, N, F = x.shape
    G = wf.shape[-1]

    def body(afw_ref, abw_ref, x_ref, wf_ref, wb_ref, out_ref):
        xb = x_ref[0]
        xwf = jnp.dot(xb, wf_ref[...], preferred_element_type=jnp.float32)
        xwb = jnp.dot(xb, wb_ref[...], preferred_element_type=jnp.float32)
        hf = jnp.maximum(
            jnp.dot(afw_ref[0], xwf, preferred_element_type=jnp.float32), 0.0)
        hb = jnp.maximum(
            jnp.dot(abw_ref[0], xwb, preferred_element_type=jnp.float32), 0.0)
        out_ref[0, 0, :] = jnp.sum(hf + hb, axis=0)

    return pl.pallas_call(
        body,
        grid=(B,),
        in_specs=[
            pl.BlockSpec((1, N, N), lambda b: (b, 0, 0)),
            pl.BlockSpec((1, N, N), lambda b: (b, 0, 0)),
            pl.BlockSpec((1, N, F), lambda b: (b, 0, 0)),
            pl.BlockSpec((F, G), lambda b: (0, 0)),
            pl.BlockSpec((F, G), lambda b: (0, 0)),
        ],
        out_specs=pl.BlockSpec((1, 1, G), lambda b: (b, 0, 0)),
        out_shape=jax.ShapeDtypeStruct((B, 1, G), jnp.float32),
        compiler_params=pltpu.CompilerParams(
            dimension_semantics=("arbitrary",)),
    )(afw, abw, x, wf, wb)


def _tc_combine(p0, p1, dsum, wsim, wcls, bcls):
    B, N, F = p0.shape
    G = wsim.shape[-1]
    O = wcls.shape[-1]
    inv_n = 1.0 / N

    def body(p0_ref, p1_ref, dsum_ref, wsim_ref, wcls_ref, b_ref, out_ref):
        agg = p0_ref[0] + p1_ref[0]
        sim = jnp.maximum(
            jnp.dot(agg, wsim_ref[...], preferred_element_type=jnp.float32),
            0.0)
        tot = (jnp.sum(sim, axis=0) + dsum_ref[0, 0]) * inv_n
        logits = jnp.dot(tot.reshape(1, G), wcls_ref[...],
                         preferred_element_type=jnp.float32) + b_ref[...]
        m = jnp.max(logits, axis=-1, keepdims=True)
        e = jnp.exp(logits - m)
        out_ref[0] = e / jnp.sum(e, axis=-1, keepdims=True)

    return pl.pallas_call(
        body,
        grid=(B,),
        in_specs=[
            pl.BlockSpec((1, N, F), lambda b: (b, 0, 0)),
            pl.BlockSpec((1, N, F), lambda b: (b, 0, 0)),
            pl.BlockSpec((1, 1, G), lambda b: (b, 0, 0)),
            pl.BlockSpec((F, G), lambda b: (0, 0)),
            pl.BlockSpec((G, O), lambda b: (0, 0)),
            pl.BlockSpec((1, O), lambda b: (0, 0)),
        ],
        out_specs=pl.BlockSpec((1, 1, O), lambda b: (b, 0, 0)),
        out_shape=jax.ShapeDtypeStruct((B, 1, O), jnp.float32),
        compiler_params=pltpu.CompilerParams(
            dimension_semantics=("arbitrary",)),
    )(p0, p1, dsum, wsim, wcls, bcls)


def kernel(true_batch_size, node_list, edge_index, fw_adj_matrix,
           bw_adj_matrix, W_sim, W_fw, W_bw, W_cls, b_cls):
    B, N, F = node_list.shape
    E = edge_index.shape[1]
    flat = node_list.reshape(B * N, F)
    eidx3 = edge_index.reshape(2, E // _EDGE_CHUNK, _EDGE_CHUNK)

    partials = _sc_scatter_partials(flat, eidx3)
    dsum = _tc_dense(fw_adj_matrix, bw_adj_matrix, node_list, W_fw, W_bw)

    p = partials.reshape(_NUM_SC, B, N, F)
    out = _tc_combine(p[0], p[1], dsum, W_sim, W_cls,
                      b_cls.reshape(1, -1))
    return out.reshape(B, -1)


# speedup vs baseline: 31.9497x; 1.7752x over previous
import functools

import jax
import jax.numpy as jnp
from jax import lax
from jax.experimental import pallas as pl
from jax.experimental.pallas import tpu as pltpu
from jax.experimental.pallas import tpu_sc as plsc

_EDGE_CHUNK = 128
_NUM_SC = 2
_NUM_TILES = 16


def _sc_scatter_partials(flat, eidx3):
    BN, F = flat.shape
    _, nchunks_total, CH = eidx3.shape
    NW = _NUM_SC * _NUM_TILES
    chunks_per_worker = nchunks_total // NW
    rows_per_tile = BN // _NUM_TILES

    mesh = plsc.VectorSubcoreMesh(core_axis_name="c", subcore_axis_name="s")

    GRP = 8
    halfchunks = chunks_per_worker // 2
    ngroups = halfchunks // GRP

    @functools.partial(
        pl.kernel,
        out_type=jax.ShapeDtypeStruct((_NUM_SC, BN, F), jnp.float32),
        mesh=mesh,
        scratch_types=[
            pltpu.VMEM((halfchunks, CH), jnp.int32),
            pltpu.VMEM((halfchunks, CH), jnp.int32),
            pltpu.VMEM((2 * GRP, CH, F), jnp.float32),
            pltpu.VMEM((CH, F), jnp.float32),
            pltpu.VMEM_SHARED((BN, F), jnp.float32),
            pltpu.SemaphoreType.DMA,
            pltpu.SemaphoreType.DMA,
        ],
        compiler_params=pltpu.CompilerParams(use_tc_tiling_on_sc=False),
    )
    def k(flat_hbm, eidx_hbm, out_hbm, src_v, dst_v, bufs, zbuf, acc,
          gsem, ssem):
        c = lax.axis_index("c")
        s = lax.axis_index("s")
        wid = c * _NUM_TILES + s

        @pl.loop(0, CH)
        def _(i):
            zbuf[i, :] = jnp.zeros((F,), jnp.float32)

        row0 = s * rows_per_tile

        @pl.loop(0, rows_per_tile // CH)
        def _(i):
            pltpu.sync_copy(zbuf, acc.at[pl.ds(row0 + i * CH, CH)])

        plsc.subcore_barrier()

        def gather(j, slot):
            pltpu.async_copy(flat_hbm.at[src_v.at[j]], bufs.at[slot], gsem)

        def gwait(j, slot):
            pltpu.make_async_copy(
                flat_hbm.at[src_v.at[j]], bufs.at[slot], gsem).wait()

        def scat(j, slot):
            pltpu.async_copy(bufs.at[slot], acc.at[dst_v.at[j]], ssem,
                             add=True)

        def swait(j, slot):
            pltpu.make_async_copy(
                bufs.at[slot], acc.at[dst_v.at[j]], ssem).wait()

        @pl.loop(0, 2)
        def _(h):
            cbase = wid * chunks_per_worker + h * halfchunks
            pltpu.sync_copy(eidx_hbm.at[0, pl.ds(cbase, halfchunks)], src_v)
            pltpu.sync_copy(eidx_hbm.at[1, pl.ds(cbase, halfchunks)], dst_v)

            for b in range(GRP):
                gather(b, b)

            @pl.loop(0, ngroups, step=2)
            def _(g):
                ja = g * GRP
                jb = ja + GRP
                for b in range(GRP):
                    gwait(ja + b, b)
                @pl.when(g > 0)
                def _():
                    for b in range(GRP):
                        swait(jb - 2 * GRP + b, GRP + b)
                for b in range(GRP):
                    gather(jb + b, GRP + b)
                for b in range(GRP):
                    scat(ja + b, b)
                for b in range(GRP):
                    gwait(jb + b, GRP + b)
                for b in range(GRP):
                    swait(ja + b, b)

                @pl.when(g + 2 < ngroups)
                def _():
                    for b in range(GRP):
                        gather(jb + GRP + b, b)

                for b in range(GRP):
                    scat(jb + b, GRP + b)

            for b in range(GRP):
                swait((ngroups - 1) * GRP + b, GRP + b)

        plsc.subcore_barrier()
        pltpu.sync_copy(acc.at[pl.ds(row0, rows_per_tile)],
                        out_hbm.at[c, pl.ds(row0, rows_per_tile)])

    return k(flat, eidx3)


def _tc_dense(afw, abw, x, wf, wb):
    B, N, F = x.shape
    G = wf.shape[-1]

    def body(afw_ref, abw_ref, x_ref, wf_ref, wb_ref, out_ref):
        xb = x_ref[0]
        xwf = jnp.dot(xb, wf_ref[...], preferred_element_type=jnp.float32)
        xwb = jnp.dot(xb, wb_ref[...], preferred_element_type=jnp.float32)
        hf = jnp.maximum(
            jnp.dot(afw_ref[0], xwf, preferred_element_type=jnp.float32), 0.0)
        hb = jnp.maximum(
            jnp.dot(abw_ref[0], xwb, preferred_element_type=jnp.float32), 0.0)
        out_ref[0, 0, :] = jnp.sum(hf + hb, axis=0)

    return pl.pallas_call(
        body,
        grid=(B,),
        in_specs=[
            pl.BlockSpec((1, N, N), lambda b: (b, 0, 0)),
            pl.BlockSpec((1, N, N), lambda b: (b, 0, 0)),
            pl.BlockSpec((1, N, F), lambda b: (b, 0, 0)),
            pl.BlockSpec((F, G), lambda b: (0, 0)),
            pl.BlockSpec((F, G), lambda b: (0, 0)),
        ],
        out_specs=pl.BlockSpec((1, 1, G), lambda b: (b, 0, 0)),
        out_shape=jax.ShapeDtypeStruct((B, 1, G), jnp.float32),
        compiler_params=pltpu.CompilerParams(
            dimension_semantics=("arbitrary",)),
    )(afw, abw, x, wf, wb)


def _tc_combine(p0, p1, dsum, wsim, wcls, bcls):
    B, N, F = p0.shape
    G = wsim.shape[-1]
    O = wcls.shape[-1]
    inv_n = 1.0 / N
    BB = 8

    def body(p0_ref, p1_ref, dsum_ref, wsim_ref, wcls_ref, b_ref, out_ref):
        agg = (p0_ref[...] + p1_ref[...]).reshape(BB * N, F)
        sim = jnp.maximum(
            jnp.dot(agg, wsim_ref[...], preferred_element_type=jnp.float32),
            0.0)
        sums = jnp.concatenate(
            [jnp.sum(sim[b * N:(b + 1) * N], axis=0, keepdims=True)
             for b in range(BB)], axis=0)
        tot = (sums + dsum_ref[...]) * inv_n
        logits = jnp.dot(tot, wcls_ref[...],
                         preferred_element_type=jnp.float32) + b_ref[...]
        m = jnp.max(logits, axis=-1, keepdims=True)
        e = jnp.exp(logits - m)
        out_ref[...] = e / jnp.sum(e, axis=-1, keepdims=True)

    return pl.pallas_call(
        body,
        grid=(B // BB,),
        in_specs=[
            pl.BlockSpec((BB, N, F), lambda b: (b, 0, 0)),
            pl.BlockSpec((BB, N, F), lambda b: (b, 0, 0)),
            pl.BlockSpec((BB, G), lambda b: (b, 0)),
            pl.BlockSpec((F, G), lambda b: (0, 0)),
            pl.BlockSpec((G, O), lambda b: (0, 0)),
            pl.BlockSpec((1, O), lambda b: (0, 0)),
        ],
        out_specs=pl.BlockSpec((BB, O), lambda b: (b, 0)),
        out_shape=jax.ShapeDtypeStruct((B, O), jnp.float32),
        compiler_params=pltpu.CompilerParams(
            dimension_semantics=("arbitrary",)),
    )(p0, p1, dsum, wsim, wcls, bcls)


def kernel(true_batch_size, node_list, edge_index, fw_adj_matrix,
           bw_adj_matrix, W_sim, W_fw, W_bw, W_cls, b_cls):
    B, N, F = node_list.shape
    E = edge_index.shape[1]
    flat = node_list.reshape(B * N, F)
    eidx3 = edge_index.reshape(2, E // _EDGE_CHUNK, _EDGE_CHUNK)

    dsum = _tc_dense(fw_adj_matrix, bw_adj_matrix, node_list, W_fw, W_bw)
    partials = _sc_scatter_partials(flat, eidx3)

    p = partials.reshape(_NUM_SC, B, N, F)
    return _tc_combine(p[0], p[1], dsum.reshape(B, -1), W_sim, W_cls,
                       b_cls.reshape(1, -1))


# speedup vs baseline: 39.9705x; 1.2510x over previous
import functools

import jax
import jax.numpy as jnp
from jax import lax
from jax.experimental import pallas as pl
from jax.experimental.pallas import tpu as pltpu
from jax.experimental.pallas import tpu_sc as plsc

_EDGE_CHUNK = 128
_NUM_SC = 2
_NUM_TILES = 16


def _sc_scatter_partials(flat, eidx3):
    BN, F = flat.shape
    _, nchunks_total, CH = eidx3.shape
    NW = _NUM_SC * _NUM_TILES
    chunks_per_worker = nchunks_total // NW
    rows_per_tile = BN // _NUM_TILES

    mesh = plsc.VectorSubcoreMesh(core_axis_name="c", subcore_axis_name="s")

    GRP = 8
    halfchunks = chunks_per_worker // 2
    ngroups = halfchunks // GRP

    @functools.partial(
        pl.kernel,
        out_type=jax.ShapeDtypeStruct((_NUM_SC, BN, F), jnp.float32),
        mesh=mesh,
        scratch_types=[
            pltpu.VMEM((halfchunks, CH), jnp.int32),
            pltpu.VMEM((halfchunks, CH), jnp.int32),
            pltpu.VMEM((2 * GRP, CH, F), jnp.float32),
            pltpu.VMEM((CH, F), jnp.float32),
            pltpu.VMEM_SHARED((BN, F), jnp.float32),
            pltpu.SemaphoreType.DMA,
            pltpu.SemaphoreType.DMA,
        ],
        compiler_params=pltpu.CompilerParams(use_tc_tiling_on_sc=False),
    )
    def k(flat_hbm, eidx_hbm, out_hbm, src_v, dst_v, bufs, zbuf, acc,
          gsem, ssem):
        c = lax.axis_index("c")
        s = lax.axis_index("s")
        wid = c * _NUM_TILES + s

        @pl.loop(0, CH)
        def _(i):
            zbuf[i, :] = jnp.zeros((F,), jnp.float32)

        row0 = s * rows_per_tile

        @pl.loop(0, rows_per_tile // CH)
        def _(i):
            pltpu.sync_copy(zbuf, acc.at[pl.ds(row0 + i * CH, CH)])

        plsc.subcore_barrier()

        def gather(j, slot):
            pltpu.async_copy(flat_hbm.at[src_v.at[j]], bufs.at[slot], gsem)

        def gwait(j, slot):
            pltpu.make_async_copy(
                flat_hbm.at[src_v.at[j]], bufs.at[slot], gsem).wait()

        def scat(j, slot):
            pltpu.async_copy(bufs.at[slot], acc.at[dst_v.at[j]], ssem,
                             add=True)

        def swait(j, slot):
            pltpu.make_async_copy(
                bufs.at[slot], acc.at[dst_v.at[j]], ssem).wait()

        @pl.loop(0, 2)
        def _(h):
            cbase = wid * chunks_per_worker + h * halfchunks
            pltpu.sync_copy(eidx_hbm.at[0, pl.ds(cbase, halfchunks)], src_v)
            pltpu.sync_copy(eidx_hbm.at[1, pl.ds(cbase, halfchunks)], dst_v)

            for b in range(GRP):
                gather(b, b)

            @pl.loop(0, ngroups, step=2)
            def _(g):
                ja = g * GRP
                jb = ja + GRP
                for b in range(GRP):
                    gwait(ja + b, b)
                @pl.when(g > 0)
                def _():
                    for b in range(GRP):
                        swait(jb - 2 * GRP + b, GRP + b)
                for b in range(GRP):
                    gather(jb + b, GRP + b)
                for b in range(GRP):
                    scat(ja + b, b)
                for b in range(GRP):
                    gwait(jb + b, GRP + b)
                for b in range(GRP):
                    swait(ja + b, b)

                @pl.when(g + 2 < ngroups)
                def _():
                    for b in range(GRP):
                        gather(jb + GRP + b, b)

                for b in range(GRP):
                    scat(jb + b, GRP + b)

            for b in range(GRP):
                swait((ngroups - 1) * GRP + b, GRP + b)

        plsc.subcore_barrier()
        pltpu.sync_copy(acc.at[pl.ds(row0, rows_per_tile)],
                        out_hbm.at[c, pl.ds(row0, rows_per_tile)])

    return k(flat, eidx3)


def _tc_dense(afw, abw, x, wf, wb):
    B, N, F = x.shape
    G = wf.shape[-1]

    def body(afw_ref, abw_ref, x_ref, wf_ref, wb_ref, out_ref):
        xb = x_ref[0]
        xwf = jnp.dot(xb, wf_ref[...],
                      preferred_element_type=jnp.float32).astype(jnp.bfloat16)
        xwb = jnp.dot(xb, wb_ref[...],
                      preferred_element_type=jnp.float32).astype(jnp.bfloat16)
        hf = jnp.maximum(
            jnp.dot(afw_ref[0].astype(jnp.bfloat16), xwf,
                    preferred_element_type=jnp.float32), 0.0)
        hb = jnp.maximum(
            jnp.dot(abw_ref[0].astype(jnp.bfloat16), xwb,
                    preferred_element_type=jnp.float32), 0.0)
        out_ref[0, 0, :] = jnp.sum(hf + hb, axis=0)

    return pl.pallas_call(
        body,
        grid=(B,),
        in_specs=[
            pl.BlockSpec((1, N, N), lambda b: (b, 0, 0)),
            pl.BlockSpec((1, N, N), lambda b: (b, 0, 0)),
            pl.BlockSpec((1, N, F), lambda b: (b, 0, 0)),
            pl.BlockSpec((F, G), lambda b: (0, 0)),
            pl.BlockSpec((F, G), lambda b: (0, 0)),
        ],
        out_specs=pl.BlockSpec((1, 1, G), lambda b: (b, 0, 0)),
        out_shape=jax.ShapeDtypeStruct((B, 1, G), jnp.float32),
        compiler_params=pltpu.CompilerParams(
            dimension_semantics=("arbitrary",)),
    )(afw, abw, x, wf, wb)


def _tc_combine(partials, dsum, wsim, wcls, bcls, n_nodes):
    _, BN, F = partials.shape
    N = n_nodes
    B = BN // N
    G = wsim.shape[-1]
    O = wcls.shape[-1]
    inv_n = 1.0 / N
    BB = 8

    def body(p_ref, dsum_ref, wsim_ref, wcls_ref, b_ref, out_ref):
        agg = p_ref[0] + p_ref[1]
        sim = jnp.maximum(
            jnp.dot(agg, wsim_ref[...], preferred_element_type=jnp.float32),
            0.0)
        sums = jnp.concatenate(
            [jnp.sum(sim[b * N:(b + 1) * N], axis=0, keepdims=True)
             for b in range(BB)], axis=0)
        tot = (sums + dsum_ref[:, 0, :]) * inv_n
        logits = jnp.dot(tot, wcls_ref[...],
                         preferred_element_type=jnp.float32) + b_ref[...]
        m = jnp.max(logits, axis=-1, keepdims=True)
        e = jnp.exp(logits - m)
        out_ref[...] = e / jnp.sum(e, axis=-1, keepdims=True)

    return pl.pallas_call(
        body,
        grid=(B // BB,),
        in_specs=[
            pl.BlockSpec((2, BB * N, F), lambda b: (0, b, 0)),
            pl.BlockSpec((BB, 1, G), lambda b: (b, 0, 0)),
            pl.BlockSpec((F, G), lambda b: (0, 0)),
            pl.BlockSpec((G, O), lambda b: (0, 0)),
            pl.BlockSpec((1, O), lambda b: (0, 0)),
        ],
        out_specs=pl.BlockSpec((BB, O), lambda b: (b, 0)),
        out_shape=jax.ShapeDtypeStruct((B, O), jnp.float32),
        compiler_params=pltpu.CompilerParams(
            dimension_semantics=("arbitrary",)),
    )(partials, dsum, wsim, wcls, bcls)


def kernel(true_batch_size, node_list, edge_index, fw_adj_matrix,
           bw_adj_matrix, W_sim, W_fw, W_bw, W_cls, b_cls):
    B, N, F = node_list.shape
    E = edge_index.shape[1]
    flat = node_list.reshape(B * N, F)
    eidx3 = edge_index.reshape(2, E // _EDGE_CHUNK, _EDGE_CHUNK)

    dsum = _tc_dense(fw_adj_matrix, bw_adj_matrix, node_list, W_fw, W_bw)
    partials = _sc_scatter_partials(flat, eidx3)
    return _tc_combine(partials, dsum, W_sim, W_cls,
                       b_cls.reshape(1, -1), N)


# speedup vs baseline: 46.3493x; 1.1596x over previous
import functools

import jax
import jax.numpy as jnp
from jax import lax
from jax.experimental import pallas as pl
from jax.experimental.pallas import tpu as pltpu
from jax.experimental.pallas import tpu_sc as plsc

_EDGE_CHUNK = 128
_NUM_SC = 2
_NUM_TILES = 16


def _sc_scatter_partials(flat, eidx3):
    BN, F = flat.shape
    _, nchunks_total, CH = eidx3.shape
    NW = _NUM_SC * _NUM_TILES
    chunks_per_worker = nchunks_total // NW
    rows_per_tile = BN // _NUM_TILES

    mesh = plsc.VectorSubcoreMesh(core_axis_name="c", subcore_axis_name="s")

    GRP = 8
    halfchunks = chunks_per_worker // 2
    ngroups = halfchunks // GRP

    NPR = 128 // F
    out_rows = BN // NPR

    @functools.partial(
        pl.kernel,
        out_type=jax.ShapeDtypeStruct((_NUM_SC, out_rows, 128), jnp.float32),
        mesh=mesh,
        scratch_types=[
            pltpu.VMEM((halfchunks, CH), jnp.int32),
            pltpu.VMEM((halfchunks, CH), jnp.int32),
            pltpu.VMEM((2 * GRP, CH, F), jnp.float32),
            pltpu.VMEM((CH, F), jnp.float32),
            pltpu.VMEM((CH, F), jnp.float32),
            pltpu.VMEM((CH // NPR, 128), jnp.float32),
            pltpu.VMEM_SHARED((BN, F), jnp.float32),
            pltpu.SemaphoreType.DMA,
            pltpu.SemaphoreType.DMA,
        ],
        compiler_params=pltpu.CompilerParams(use_tc_tiling_on_sc=False),
    )
    def k(flat_hbm, eidx_hbm, out_hbm, src_v, dst_v, bufs, zbuf, wbuf, wbuf2,
          acc, gsem, ssem):
        c = lax.axis_index("c")
        s = lax.axis_index("s")
        wid = c * _NUM_TILES + s

        @pl.loop(0, CH)
        def _(i):
            zbuf[i, :] = jnp.zeros((F,), jnp.float32)

        row0 = s * rows_per_tile

        @pl.loop(0, rows_per_tile // CH)
        def _(i):
            pltpu.sync_copy(zbuf, acc.at[pl.ds(row0 + i * CH, CH)])

        plsc.subcore_barrier()

        def gather(j, slot):
            pltpu.async_copy(flat_hbm.at[src_v.at[j]], bufs.at[slot], gsem)

        def gwait(j, slot):
            pltpu.make_async_copy(
                flat_hbm.at[src_v.at[j]], bufs.at[slot], gsem).wait()

        def scat(j, slot):
            pltpu.async_copy(bufs.at[slot], acc.at[dst_v.at[j]], ssem,
                             add=True)

        def swait(j, slot):
            pltpu.make_async_copy(
                bufs.at[slot], acc.at[dst_v.at[j]], ssem).wait()

        @pl.loop(0, 2)
        def _(h):
            cbase = wid * chunks_per_worker + h * halfchunks
            pltpu.sync_copy(eidx_hbm.at[0, pl.ds(cbase, halfchunks)], src_v)
            pltpu.sync_copy(eidx_hbm.at[1, pl.ds(cbase, halfchunks)], dst_v)

            for b in range(GRP):
                gather(b, b)

            @pl.loop(0, ngroups, step=2)
            def _(g):
                ja = g * GRP
                jb = ja + GRP
                for b in range(GRP):
                    gwait(ja + b, b)
                @pl.when(g > 0)
                def _():
                    for b in range(GRP):
                        swait(jb - 2 * GRP + b, GRP + b)
                for b in range(GRP):
                    gather(jb + b, GRP + b)
                for b in range(GRP):
                    scat(ja + b, b)
                for b in range(GRP):
                    gwait(jb + b, GRP + b)
                for b in range(GRP):
                    swait(ja + b, b)

                @pl.when(g + 2 < ngroups)
                def _():
                    for b in range(GRP):
                        gather(jb + GRP + b, b)

                for b in range(GRP):
                    scat(jb + b, GRP + b)

            for b in range(GRP):
                swait((ngroups - 1) * GRP + b, GRP + b)

        plsc.subcore_barrier()

        @pl.loop(0, rows_per_tile // CH)
        def _(kblk):
            pltpu.sync_copy(acc.at[pl.ds(row0 + kblk * CH, CH)], wbuf)
            for j in range(CH):
                wbuf2[j // NPR, pl.ds((j % NPR) * F, F)] = wbuf[j, :]
            pltpu.sync_copy(
                wbuf2,
                out_hbm.at[c, pl.ds(s * (rows_per_tile // NPR)
                                    + kblk * (CH // NPR), CH // NPR)])

    return k(flat, eidx3)


def _tc_dense(afw, abw, x, wf, wb):
    B, N, F = x.shape
    G = wf.shape[-1]

    def body(afw_ref, abw_ref, x_ref, wf_ref, wb_ref, out_ref):
        xb = x_ref[0]
        xwf = jnp.dot(xb, wf_ref[...],
                      preferred_element_type=jnp.float32).astype(jnp.bfloat16)
        xwb = jnp.dot(xb, wb_ref[...],
                      preferred_element_type=jnp.float32).astype(jnp.bfloat16)
        hf = jnp.maximum(
            jnp.dot(afw_ref[0].astype(jnp.bfloat16), xwf,
                    preferred_element_type=jnp.float32), 0.0)
        hb = jnp.maximum(
            jnp.dot(abw_ref[0].astype(jnp.bfloat16), xwb,
                    preferred_element_type=jnp.float32), 0.0)
        out_ref[0, 0, :] = jnp.sum(hf + hb, axis=0)

    return pl.pallas_call(
        body,
        grid=(B,),
        in_specs=[
            pl.BlockSpec((1, N, N), lambda b: (b, 0, 0)),
            pl.BlockSpec((1, N, N), lambda b: (b, 0, 0)),
            pl.BlockSpec((1, N, F), lambda b: (b, 0, 0)),
            pl.BlockSpec((F, G), lambda b: (0, 0)),
            pl.BlockSpec((F, G), lambda b: (0, 0)),
        ],
        out_specs=pl.BlockSpec((1, 1, G), lambda b: (b, 0, 0)),
        out_shape=jax.ShapeDtypeStruct((B, 1, G), jnp.float32),
        compiler_params=pltpu.CompilerParams(
            dimension_semantics=("arbitrary",)),
    )(afw, abw, x, wf, wb)


def _tc_combine(partials, dsum, wsim_diag, wcls, bcls, n_nodes, n_feat):
    _, out_rows, _ = partials.shape
    N = n_nodes
    F = n_feat
    NPR = 128 // F
    G = wsim_diag.shape[-1] // NPR
    B = out_rows * NPR // N
    O = wcls.shape[-1]
    inv_n = 1.0 / N
    BB = 8
    RPB = N // NPR

    def body(p_ref, dsum_ref, wsim_ref, wcls_ref, b_ref, out_ref):
        agg = p_ref[0] + p_ref[1]
        sim = jnp.maximum(
            jnp.dot(agg, wsim_ref[...], preferred_element_type=jnp.float32),
            0.0)
        rows = []
        for b in range(BB):
            v = jnp.sum(sim[b * RPB:(b + 1) * RPB], axis=0)
            acc = v[0:G]
            for j in range(1, NPR):
                acc = acc + v[j * G:(j + 1) * G]
            rows.append(acc.reshape(1, G))
        sums = jnp.concatenate(rows, axis=0)
        tot = (sums + dsum_ref[:, 0, :]) * inv_n
        logits = jnp.dot(tot, wcls_ref[...],
                         preferred_element_type=jnp.float32) + b_ref[...]
        m = jnp.max(logits, axis=-1, keepdims=True)
        e = jnp.exp(logits - m)
        out_ref[...] = e / jnp.sum(e, axis=-1, keepdims=True)

    return pl.pallas_call(
        body,
        grid=(B // BB,),
        in_specs=[
            pl.BlockSpec((2, BB * RPB, 128), lambda b: (0, b, 0)),
            pl.BlockSpec((BB, 1, G), lambda b: (b, 0, 0)),
            pl.BlockSpec((128, NPR * G), lambda b: (0, 0)),
            pl.BlockSpec((G, O), lambda b: (0, 0)),
            pl.BlockSpec((1, O), lambda b: (0, 0)),
        ],
        out_specs=pl.BlockSpec((BB, O), lambda b: (b, 0)),
        out_shape=jax.ShapeDtypeStruct((B, O), jnp.float32),
        compiler_params=pltpu.CompilerParams(
            dimension_semantics=("arbitrary",)),
    )(partials, dsum, wsim_diag, wcls, bcls)


def kernel(true_batch_size, node_list, edge_index, fw_adj_matrix,
           bw_adj_matrix, W_sim, W_fw, W_bw, W_cls, b_cls):
    B, N, F = node_list.shape
    E = edge_index.shape[1]
    flat = node_list.reshape(B * N, F)
    eidx3 = edge_index.reshape(2, E // _EDGE_CHUNK, _EDGE_CHUNK)

    dsum = _tc_dense(fw_adj_matrix, bw_adj_matrix, node_list, W_fw, W_bw)
    partials = _sc_scatter_partials(flat, eidx3)
    wsim_diag = jnp.kron(jnp.eye(128 // F, dtype=W_sim.dtype), W_sim)
    return _tc_combine(partials, dsum, wsim_diag, W_cls,
                       b_cls.reshape(1, -1), N, F)


# speedup vs baseline: 49.2796x; 1.0632x over previous
import functools

import jax
import jax.numpy as jnp
from jax import lax
from jax.experimental import pallas as pl
from jax.experimental.pallas import tpu as pltpu
from jax.experimental.pallas import tpu_sc as plsc

_EDGE_CHUNK = 128
_NUM_SC = 2
_NUM_TILES = 16


def _sc_scatter_partials(flat128, eidx3, n_feat):
    F = n_feat
    BN = flat128.shape[0] * 128 // F
    _, nchunks_total, CH = eidx3.shape
    NW = _NUM_SC * _NUM_TILES
    chunks_per_worker = nchunks_total // NW
    rows_per_tile = BN // _NUM_TILES

    mesh = plsc.VectorSubcoreMesh(core_axis_name="c", subcore_axis_name="s")

    GRP = 4
    halfchunks = chunks_per_worker // 2
    ngroups = halfchunks // GRP

    NPR = 128 // F
    out_rows = BN // NPR

    @functools.partial(
        pl.kernel,
        out_type=jax.ShapeDtypeStruct((_NUM_SC, out_rows, 128), jnp.float32),
        mesh=mesh,
        scratch_types=[
            pltpu.VMEM((halfchunks, CH), jnp.int32),
            pltpu.VMEM((halfchunks, CH), jnp.int32),
            pltpu.VMEM((2 * GRP, CH, F), jnp.float32),
            pltpu.VMEM((CH, F), jnp.float32),
            pltpu.VMEM((CH, F), jnp.float32),
            pltpu.VMEM((CH // NPR, 128), jnp.float32),
            pltpu.VMEM_SHARED((BN, F), jnp.float32),
            pltpu.VMEM_SHARED((BN, F), jnp.float32),
            pltpu.SemaphoreType.DMA,
            pltpu.SemaphoreType.DMA,
        ],
        compiler_params=pltpu.CompilerParams(use_tc_tiling_on_sc=False),
    )
    def k(flat_hbm, eidx_hbm, out_hbm, src_v, dst_v, bufs, zbuf, wbuf, wbuf2,
          acc, table, gsem, ssem):
        c = lax.axis_index("c")
        s = lax.axis_index("s")
        wid = c * _NUM_TILES + s

        @pl.loop(0, CH)
        def _(i):
            zbuf[i, :] = jnp.zeros((F,), jnp.float32)

        row0 = s * rows_per_tile

        @pl.loop(0, rows_per_tile // CH)
        def _(i):
            pltpu.sync_copy(zbuf, acc.at[pl.ds(row0 + i * CH, CH)])

        @pl.loop(0, rows_per_tile // CH)
        def _(kblk):
            pltpu.sync_copy(
                flat_hbm.at[pl.ds(s * (rows_per_tile // NPR)
                                  + kblk * (CH // NPR), CH // NPR)], wbuf2)
            for j in range(CH):
                wbuf[j, :] = wbuf2[j // NPR, pl.ds((j % NPR) * F, F)]
            pltpu.sync_copy(wbuf, table.at[pl.ds(row0 + kblk * CH, CH)])

        plsc.subcore_barrier()

        def gather(j, slot):
            pltpu.async_copy(table.at[src_v.at[j]], bufs.at[slot], gsem)

        def gwait(j, slot):
            pltpu.make_async_copy(
                table.at[src_v.at[j]], bufs.at[slot], gsem).wait()

        def scat(j, slot):
            pltpu.async_copy(bufs.at[slot], acc.at[dst_v.at[j]], ssem,
                             add=True)

        def swait(j, slot):
            pltpu.make_async_copy(
                bufs.at[slot], acc.at[dst_v.at[j]], ssem).wait()

        @pl.loop(0, 2)
        def _(h):
            cbase = wid * chunks_per_worker + h * halfchunks
            pltpu.sync_copy(eidx_hbm.at[0, pl.ds(cbase, halfchunks)], src_v)
            pltpu.sync_copy(eidx_hbm.at[1, pl.ds(cbase, halfchunks)], dst_v)

            for b in range(GRP):
                gather(b, b)

            @pl.loop(0, ngroups, step=2)
            def _(g):
                ja = g * GRP
                jb = ja + GRP
                for b in range(GRP):
                    gwait(ja + b, b)
                @pl.when(g > 0)
                def _():
                    for b in range(GRP):
                        swait(jb - 2 * GRP + b, GRP + b)
                for b in range(GRP):
                    gather(jb + b, GRP + b)
                for b in range(GRP):
                    scat(ja + b, b)
                for b in range(GRP):
                    gwait(jb + b, GRP + b)
                for b in range(GRP):
                    swait(ja + b, b)

                @pl.when(g + 2 < ngroups)
                def _():
                    for b in range(GRP):
                        gather(jb + GRP + b, b)

                for b in range(GRP):
                    scat(jb + b, GRP + b)

            for b in range(GRP):
                swait((ngroups - 1) * GRP + b, GRP + b)

        plsc.subcore_barrier()

        @pl.loop(0, rows_per_tile // CH)
        def _(kblk):
            pltpu.sync_copy(acc.at[pl.ds(row0 + kblk * CH, CH)], wbuf)
            for j in range(CH):
                wbuf2[j // NPR, pl.ds((j % NPR) * F, F)] = wbuf[j, :]
            pltpu.sync_copy(
                wbuf2,
                out_hbm.at[c, pl.ds(s * (rows_per_tile // NPR)
                                    + kblk * (CH // NPR), CH // NPR)])

    return k(flat128, eidx3)


def _tc_dense(afw, abw, x, wf, wb):
    B, N, F = x.shape
    G = wf.shape[-1]

    def body(afw_ref, abw_ref, x_ref, wf_ref, wb_ref, out_ref):
        xb = x_ref[0]
        xwf = jnp.dot(xb, wf_ref[...],
                      preferred_element_type=jnp.float32).astype(jnp.bfloat16)
        xwb = jnp.dot(xb, wb_ref[...],
                      preferred_element_type=jnp.float32).astype(jnp.bfloat16)
        hf = jnp.maximum(
            jnp.dot(afw_ref[0].astype(jnp.bfloat16), xwf,
                    preferred_element_type=jnp.float32), 0.0)
        hb = jnp.maximum(
            jnp.dot(abw_ref[0].astype(jnp.bfloat16), xwb,
                    preferred_element_type=jnp.float32), 0.0)
        out_ref[0, 0, :] = jnp.sum(hf + hb, axis=0)

    return pl.pallas_call(
        body,
        grid=(B,),
        in_specs=[
            pl.BlockSpec((1, N, N), lambda b: (b, 0, 0)),
            pl.BlockSpec((1, N, N), lambda b: (b, 0, 0)),
            pl.BlockSpec((1, N, F), lambda b: (b, 0, 0)),
            pl.BlockSpec((F, G), lambda b: (0, 0)),
            pl.BlockSpec((F, G), lambda b: (0, 0)),
        ],
        out_specs=pl.BlockSpec((1, 1, G), lambda b: (b, 0, 0)),
        out_shape=jax.ShapeDtypeStruct((B, 1, G), jnp.float32),
        compiler_params=pltpu.CompilerParams(
            dimension_semantics=("arbitrary",)),
    )(afw, abw, x, wf, wb)


def _tc_combine(partials, dsum, wsim_diag, wcls, bcls, n_nodes, n_feat):
    _, out_rows, _ = partials.shape
    N = n_nodes
    F = n_feat
    NPR = 128 // F
    G = wsim_diag.shape[-1] // NPR
    B = out_rows * NPR // N
    O = wcls.shape[-1]
    inv_n = 1.0 / N
    BB = 8
    RPB = N // NPR

    def body(p_ref, dsum_ref, wsim_ref, wcls_ref, b_ref, out_ref):
        agg = p_ref[0] + p_ref[1]
        sim = jnp.maximum(
            jnp.dot(agg, wsim_ref[...], preferred_element_type=jnp.float32),
            0.0)
        rows = []
        for b in range(BB):
            v = jnp.sum(sim[b * RPB:(b + 1) * RPB], axis=0)
            acc = v[0:G]
            for j in range(1, NPR):
                acc = acc + v[j * G:(j + 1) * G]
            rows.append(acc.reshape(1, G))
        sums = jnp.concatenate(rows, axis=0)
        tot = (sums + dsum_ref[:, 0, :]) * inv_n
        logits = jnp.dot(tot, wcls_ref[...],
                         preferred_element_type=jnp.float32) + b_ref[...]
        m = jnp.max(logits, axis=-1, keepdims=True)
        e = jnp.exp(logits - m)
        out_ref[...] = e / jnp.sum(e, axis=-1, keepdims=True)

    return pl.pallas_call(
        body,
        grid=(B // BB,),
        in_specs=[
            pl.BlockSpec((2, BB * RPB, 128), lambda b: (0, b, 0)),
            pl.BlockSpec((BB, 1, G), lambda b: (b, 0, 0)),
            pl.BlockSpec((128, NPR * G), lambda b: (0, 0)),
            pl.BlockSpec((G, O), lambda b: (0, 0)),
            pl.BlockSpec((1, O), lambda b: (0, 0)),
        ],
        out_specs=pl.BlockSpec((BB, O), lambda b: (b, 0)),
        out_shape=jax.ShapeDtypeStruct((B, O), jnp.float32),
        compiler_params=pltpu.CompilerParams(
            dimension_semantics=("arbitrary",)),
    )(partials, dsum, wsim_diag, wcls, bcls)


def kernel(true_batch_size, node_list, edge_index, fw_adj_matrix,
           bw_adj_matrix, W_sim, W_fw, W_bw, W_cls, b_cls):
    B, N, F = node_list.shape
    E = edge_index.shape[1]
    eidx3 = edge_index.reshape(2, E // _EDGE_CHUNK, _EDGE_CHUNK)

    dsum = _tc_dense(fw_adj_matrix, bw_adj_matrix, node_list, W_fw, W_bw)
    flat128 = node_list.reshape(B * N * F // 128, 128)
    partials = _sc_scatter_partials(flat128, eidx3, F)
    wsim_diag = jnp.kron(jnp.eye(128 // F, dtype=W_sim.dtype), W_sim)
    return _tc_combine(partials, dsum, wsim_diag, W_cls,
                       b_cls.reshape(1, -1), N, F)


# speedup vs baseline: 49.7217x; 1.0090x over previous
import functools

import jax
import jax.numpy as jnp
from jax import lax
from jax.experimental import pallas as pl
from jax.experimental.pallas import tpu as pltpu
from jax.experimental.pallas import tpu_sc as plsc

_EDGE_CHUNK = 128
_NUM_SC = 2
_NUM_TILES = 16


def _sc_scatter_partials(flat128, edge_index, n_feat):
    F = n_feat
    BN = flat128.shape[0] * 128 // F
    CH = _EDGE_CHUNK
    E = edge_index.shape[1]
    nchunks_total = E // CH
    NW = _NUM_SC * _NUM_TILES
    chunks_per_worker = nchunks_total // NW
    rows_per_tile = BN // _NUM_TILES

    mesh = plsc.VectorSubcoreMesh(core_axis_name="c", subcore_axis_name="s")

    GRP = 4
    halfchunks = chunks_per_worker // 2
    ngroups = halfchunks // GRP

    NPR = 128 // F
    out_rows = BN // NPR

    @functools.partial(
        pl.kernel,
        out_type=jax.ShapeDtypeStruct((_NUM_SC, out_rows, 128), jnp.float32),
        mesh=mesh,
        scratch_types=[
            pltpu.VMEM((halfchunks * CH,), jnp.int32),
            pltpu.VMEM((halfchunks, CH), jnp.int32),
            pltpu.VMEM((2 * GRP, CH, F), jnp.float32),
            pltpu.VMEM((CH, F), jnp.float32),
            pltpu.VMEM((CH, F), jnp.float32),
            pltpu.VMEM((CH // NPR, 128), jnp.float32),
            pltpu.VMEM_SHARED((BN, F), jnp.float32),
            pltpu.VMEM_SHARED((BN, F), jnp.float32),
            pltpu.SemaphoreType.DMA,
            pltpu.SemaphoreType.DMA,
        ],
        compiler_params=pltpu.CompilerParams(use_tc_tiling_on_sc=False),
    )
    def k(flat_hbm, eidx_hbm, out_hbm, src_v, dst_v, bufs, zbuf, wbuf, wbuf2,
          acc, table, gsem, ssem):
        c = lax.axis_index("c")
        s = lax.axis_index("s")
        wid = c * _NUM_TILES + s

        @pl.loop(0, CH)
        def _(i):
            zbuf[i, :] = jnp.zeros((F,), jnp.float32)

        row0 = s * rows_per_tile

        @pl.loop(0, rows_per_tile // CH)
        def _(i):
            pltpu.sync_copy(zbuf, acc.at[pl.ds(row0 + i * CH, CH)])

        @pl.loop(0, rows_per_tile // CH)
        def _(kblk):
            pltpu.sync_copy(
                flat_hbm.at[pl.ds(s * (rows_per_tile // NPR)
                                  + kblk * (CH // NPR), CH // NPR)], wbuf2)
            for j in range(CH):
                wbuf[j, :] = wbuf2[j // NPR, pl.ds((j % NPR) * F, F)]
            pltpu.sync_copy(wbuf, table.at[pl.ds(row0 + kblk * CH, CH)])

        plsc.subcore_barrier()

        def gather(j, slot):
            pltpu.async_copy(
                table.at[src_v.at[pl.ds(j * CH, CH)]], bufs.at[slot], gsem)

        def gwait(j, slot):
            pltpu.make_async_copy(
                table.at[src_v.at[pl.ds(j * CH, CH)]], bufs.at[slot],
                gsem).wait()

        def scat(j, slot):
            pltpu.async_copy(bufs.at[slot], acc.at[dst_v.at[j]], ssem,
                             add=True)

        def swait(j, slot):
            pltpu.make_async_copy(
                bufs.at[slot], acc.at[dst_v.at[j]], ssem).wait()

        @pl.loop(0, 2)
        def _(h):
            ebase = (wid * chunks_per_worker + h * halfchunks) * CH
            pltpu.async_copy(eidx_hbm.at[0, pl.ds(ebase, halfchunks * CH)],
                             src_v, gsem)

            @pl.loop(0, halfchunks)
            def _(r):
                pltpu.async_copy(eidx_hbm.at[1, pl.ds(ebase + r * CH, CH)],
                                 dst_v.at[r], gsem)

            pltpu.make_async_copy(
                eidx_hbm.at[0, pl.ds(ebase, halfchunks * CH)], src_v,
                gsem).wait()

            @pl.loop(0, halfchunks)
            def _(r):
                pltpu.make_async_copy(
                    eidx_hbm.at[1, pl.ds(ebase, CH)], dst_v.at[0],
                    gsem).wait()

            for b in range(GRP):
                gather(b, b)

            @pl.loop(0, ngroups, step=2)
            def _(g):
                ja = g * GRP
                jb = ja + GRP
                for b in range(GRP):
                    gwait(ja + b, b)
                @pl.when(g > 0)
                def _():
                    for b in range(GRP):
                        swait(jb - 2 * GRP + b, GRP + b)
                for b in range(GRP):
                    gather(jb + b, GRP + b)
                for b in range(GRP):
                    scat(ja + b, b)
                for b in range(GRP):
                    gwait(jb + b, GRP + b)
                for b in range(GRP):
                    swait(ja + b, b)

                @pl.when(g + 2 < ngroups)
                def _():
                    for b in range(GRP):
                        gather(jb + GRP + b, b)

                for b in range(GRP):
                    scat(jb + b, GRP + b)

            for b in range(GRP):
                swait((ngroups - 1) * GRP + b, GRP + b)

        plsc.subcore_barrier()

        @pl.loop(0, rows_per_tile // CH)
        def _(kblk):
            pltpu.sync_copy(acc.at[pl.ds(row0 + kblk * CH, CH)], wbuf)
            for j in range(CH):
                wbuf2[j // NPR, pl.ds((j % NPR) * F, F)] = wbuf[j, :]
            pltpu.sync_copy(
                wbuf2,
                out_hbm.at[c, pl.ds(s * (rows_per_tile // NPR)
                                    + kblk * (CH // NPR), CH // NPR)])

    return k(flat128, edge_index)


def _tc_dense(afw, abw, x, wf, wb):
    B, N, F = x.shape
    G = wf.shape[-1]

    def body(afw_ref, abw_ref, x_ref, wf_ref, wb_ref, out_ref):
        xb = x_ref[0]
        xwf = jnp.dot(xb, wf_ref[...],
                      preferred_element_type=jnp.float32).astype(jnp.bfloat16)
        xwb = jnp.dot(xb, wb_ref[...],
                      preferred_element_type=jnp.float32).astype(jnp.bfloat16)
        hf = jnp.maximum(
            jnp.dot(afw_ref[0].astype(jnp.bfloat16), xwf,
                    preferred_element_type=jnp.float32), 0.0)
        hb = jnp.maximum(
            jnp.dot(abw_ref[0].astype(jnp.bfloat16), xwb,
                    preferred_element_type=jnp.float32), 0.0)
        out_ref[0, 0, :] = jnp.sum(hf + hb, axis=0)

    return pl.pallas_call(
        body,
        grid=(B,),
        in_specs=[
            pl.BlockSpec((1, N, N), lambda b: (b, 0, 0)),
            pl.BlockSpec((1, N, N), lambda b: (b, 0, 0)),
            pl.BlockSpec((1, N, F), lambda b: (b, 0, 0)),
            pl.BlockSpec((F, G), lambda b: (0, 0)),
            pl.BlockSpec((F, G), lambda b: (0, 0)),
        ],
        out_specs=pl.BlockSpec((1, 1, G), lambda b: (b, 0, 0)),
        out_shape=jax.ShapeDtypeStruct((B, 1, G), jnp.float32),
        compiler_params=pltpu.CompilerParams(
            dimension_semantics=("arbitrary",)),
    )(afw, abw, x, wf, wb)


def _tc_combine(partials, dsum, wsim_diag, wcls, bcls, n_nodes, n_feat):
    _, out_rows, _ = partials.shape
    N = n_nodes
    F = n_feat
    NPR = 128 // F
    G = wsim_diag.shape[-1] // NPR
    B = out_rows * NPR // N
    O = wcls.shape[-1]
    inv_n = 1.0 / N
    BB = 8
    RPB = N // NPR

    def body(p_ref, dsum_ref, wsim_ref, wcls_ref, b_ref, out_ref):
        agg = p_ref[0] + p_ref[1]
        sim = jnp.maximum(
            jnp.dot(agg, wsim_ref[...], preferred_element_type=jnp.float32),
            0.0)
        rows = []
        for b in range(BB):
            v = jnp.sum(sim[b * RPB:(b + 1) * RPB], axis=0)
            acc = v[0:G]
            for j in range(1, NPR):
                acc = acc + v[j * G:(j + 1) * G]
            rows.append(acc.reshape(1, G))
        sums = jnp.concatenate(rows, axis=0)
        tot = (sums + dsum_ref[:, 0, :]) * inv_n
        logits = jnp.dot(tot, wcls_ref[...],
                         preferred_element_type=jnp.float32) + b_ref[...]
        m = jnp.max(logits, axis=-1, keepdims=True)
        e = jnp.exp(logits - m)
        out_ref[...] = e / jnp.sum(e, axis=-1, keepdims=True)

    return pl.pallas_call(
        body,
        grid=(B // BB,),
        in_specs=[
            pl.BlockSpec((2, BB * RPB, 128), lambda b: (0, b, 0)),
            pl.BlockSpec((BB, 1, G), lambda b: (b, 0, 0)),
            pl.BlockSpec((128, NPR * G), lambda b: (0, 0)),
            pl.BlockSpec((G, O), lambda b: (0, 0)),
            pl.BlockSpec((1, O), lambda b: (0, 0)),
        ],
        out_specs=pl.BlockSpec((BB, O), lambda b: (b, 0)),
        out_shape=jax.ShapeDtypeStruct((B, O), jnp.float32),
        compiler_params=pltpu.CompilerParams(
            dimension_semantics=("arbitrary",)),
    )(partials, dsum, wsim_diag, wcls, bcls)


def kernel(true_batch_size, node_list, edge_index, fw_adj_matrix,
           bw_adj_matrix, W_sim, W_fw, W_bw, W_cls, b_cls):
    B, N, F = node_list.shape
    dsum = _tc_dense(fw_adj_matrix, bw_adj_matrix, node_list, W_fw, W_bw)
    flat128 = node_list.reshape(B * N * F // 128, 128)
    partials = _sc_scatter_partials(flat128, edge_index, F)
    wsim_diag = jnp.kron(jnp.eye(128 // F, dtype=W_sim.dtype), W_sim)
    return _tc_combine(partials, dsum, wsim_diag, W_cls,
                       b_cls.reshape(1, -1), N, F)


# speedup vs baseline: 51.4316x; 1.0344x over previous
import functools

import jax
import jax.numpy as jnp
from jax import lax
from jax.experimental import pallas as pl
from jax.experimental.pallas import tpu as pltpu
from jax.experimental.pallas import tpu_sc as plsc

_EDGE_CHUNK = 128
_NUM_SC = 2
_NUM_TILES = 16


def _sc_scatter_partials(node_list, edge_index):
    B, N, F = node_list.shape
    BN = B * N
    CH = _EDGE_CHUNK
    E = edge_index.shape[1]
    nchunks_total = E // CH
    NW = _NUM_SC * _NUM_TILES
    chunks_per_worker = nchunks_total // NW
    rows_per_tile = BN // _NUM_TILES

    mesh = plsc.VectorSubcoreMesh(core_axis_name="c", subcore_axis_name="s")

    GRP = 4
    halfchunks = chunks_per_worker // 2
    ngroups = halfchunks // GRP

    NPR = 128 // F
    out_rows = BN // NPR

    @functools.partial(
        pl.kernel,
        out_type=jax.ShapeDtypeStruct((_NUM_SC, out_rows, 128), jnp.float32),
        mesh=mesh,
        scratch_types=[
            pltpu.VMEM((halfchunks * CH,), jnp.int32),
            pltpu.VMEM((halfchunks, CH), jnp.int32),
            pltpu.VMEM((2 * GRP, CH, F), jnp.float32),
            pltpu.VMEM((CH, F), jnp.float32),
            pltpu.VMEM((CH, F), jnp.float32),
            pltpu.VMEM((CH // NPR, 128), jnp.float32),
            pltpu.VMEM_SHARED((BN, F), jnp.float32),
            pltpu.VMEM_SHARED((BN, F), jnp.float32),
            pltpu.SemaphoreType.DMA,
            pltpu.SemaphoreType.DMA,
        ],
        compiler_params=pltpu.CompilerParams(use_tc_tiling_on_sc=False),
    )
    def k(flat_hbm, eidx_hbm, out_hbm, src_v, dst_v, bufs, zbuf, wbuf, wbuf2,
          acc, table, gsem, ssem):
        c = lax.axis_index("c")
        s = lax.axis_index("s")
        wid = c * _NUM_TILES + s

        @pl.loop(0, CH)
        def _(i):
            zbuf[i, :] = jnp.zeros((F,), jnp.float32)

        row0 = s * rows_per_tile

        @pl.loop(0, rows_per_tile // CH)
        def _(i):
            pltpu.sync_copy(zbuf, acc.at[pl.ds(row0 + i * CH, CH)])

        bpt = rows_per_tile // N

        @pl.loop(0, bpt)
        def _(i):
            pltpu.sync_copy(flat_hbm.at[s * bpt + i],
                            table.at[pl.ds(row0 + i * N, N)])

        plsc.subcore_barrier()

        def gather(j, slot):
            pltpu.async_copy(
                table.at[src_v.at[pl.ds(j * CH, CH)]], bufs.at[slot], gsem)

        def gwait(j, slot):
            pltpu.make_async_copy(
                table.at[src_v.at[pl.ds(j * CH, CH)]], bufs.at[slot],
                gsem).wait()

        def scat(j, slot):
            pltpu.async_copy(bufs.at[slot], acc.at[dst_v.at[j]], ssem,
                             add=True)

        def swait(j, slot):
            pltpu.make_async_copy(
                bufs.at[slot], acc.at[dst_v.at[j]], ssem).wait()

        @pl.loop(0, 2)
        def _(h):
            ebase = (wid * chunks_per_worker + h * halfchunks) * CH
            pltpu.async_copy(eidx_hbm.at[0, pl.ds(ebase, halfchunks * CH)],
                             src_v, gsem)

            @pl.loop(0, halfchunks)
            def _(r):
                pltpu.async_copy(eidx_hbm.at[1, pl.ds(ebase + r * CH, CH)],
                                 dst_v.at[r], gsem)

            pltpu.make_async_copy(
                eidx_hbm.at[0, pl.ds(ebase, halfchunks * CH)], src_v,
                gsem).wait()

            @pl.loop(0, halfchunks)
            def _(r):
                pltpu.make_async_copy(
                    eidx_hbm.at[1, pl.ds(ebase, CH)], dst_v.at[0],
                    gsem).wait()

            for b in range(GRP):
                gather(b, b)

            @pl.loop(0, ngroups, step=2)
            def _(g):
                ja = g * GRP
                jb = ja + GRP
                for b in range(GRP):
                    gwait(ja + b, b)
                @pl.when(g > 0)
                def _():
                    for b in range(GRP):
                        swait(jb - 2 * GRP + b, GRP + b)
                for b in range(GRP):
                    gather(jb + b, GRP + b)
                for b in range(GRP):
                    scat(ja + b, b)
                for b in range(GRP):
                    gwait(jb + b, GRP + b)
                for b in range(GRP):
                    swait(ja + b, b)

                @pl.when(g + 2 < ngroups)
                def _():
                    for b in range(GRP):
                        gather(jb + GRP + b, b)

                for b in range(GRP):
                    scat(jb + b, GRP + b)

            for b in range(GRP):
                swait((ngroups - 1) * GRP + b, GRP + b)

        plsc.subcore_barrier()

        @pl.loop(0, rows_per_tile // CH)
        def _(kblk):
            pltpu.sync_copy(acc.at[pl.ds(row0 + kblk * CH, CH)], wbuf)
            for j in range(CH):
                wbuf2[j // NPR, pl.ds((j % NPR) * F, F)] = wbuf[j, :]
            pltpu.sync_copy(
                wbuf2,
                out_hbm.at[c, pl.ds(s * (rows_per_tile // NPR)
                                    + kblk * (CH // NPR), CH // NPR)])

    return k(node_list, edge_index)


def _tc_dense(afw, abw, x, wf, wb):
    B, N, F = x.shape
    G = wf.shape[-1]

    def body(afw_ref, abw_ref, x_ref, wf_ref, wb_ref, out_ref):
        xb = x_ref[0]
        xwf = jnp.dot(xb, wf_ref[...],
                      preferred_element_type=jnp.float32).astype(jnp.bfloat16)
        xwb = jnp.dot(xb, wb_ref[...],
                      preferred_element_type=jnp.float32).astype(jnp.bfloat16)
        hf = jnp.maximum(
            jnp.dot(afw_ref[0].astype(jnp.bfloat16), xwf,
                    preferred_element_type=jnp.float32), 0.0)
        hb = jnp.maximum(
            jnp.dot(abw_ref[0].astype(jnp.bfloat16), xwb,
                    preferred_element_type=jnp.float32), 0.0)
        out_ref[0, 0, :] = jnp.sum(hf + hb, axis=0)

    return pl.pallas_call(
        body,
        grid=(B,),
        in_specs=[
            pl.BlockSpec((1, N, N), lambda b: (b, 0, 0)),
            pl.BlockSpec((1, N, N), lambda b: (b, 0, 0)),
            pl.BlockSpec((1, N, F), lambda b: (b, 0, 0)),
            pl.BlockSpec((F, G), lambda b: (0, 0)),
            pl.BlockSpec((F, G), lambda b: (0, 0)),
        ],
        out_specs=pl.BlockSpec((1, 1, G), lambda b: (b, 0, 0)),
        out_shape=jax.ShapeDtypeStruct((B, 1, G), jnp.float32),
        compiler_params=pltpu.CompilerParams(
            dimension_semantics=("arbitrary",)),
    )(afw, abw, x, wf, wb)


def _tc_combine(partials, dsum, wsim_diag, wcls, bcls, n_nodes, n_feat):
    _, out_rows, _ = partials.shape
    N = n_nodes
    F = n_feat
    NPR = 128 // F
    G = wsim_diag.shape[-1] // NPR
    B = out_rows * NPR // N
    O = wcls.shape[-1]
    inv_n = 1.0 / N
    BB = 8
    RPB = N // NPR

    def body(p_ref, dsum_ref, wsim_ref, wcls_ref, b_ref, out_ref):
        agg = p_ref[0] + p_ref[1]
        sim = jnp.maximum(
            jnp.dot(agg, wsim_ref[...], preferred_element_type=jnp.float32),
            0.0)
        rows = []
        for b in range(BB):
            v = jnp.sum(sim[b * RPB:(b + 1) * RPB], axis=0)
            acc = v[0:G]
            for j in range(1, NPR):
                acc = acc + v[j * G:(j + 1) * G]
            rows.append(acc.reshape(1, G))
        sums = jnp.concatenate(rows, axis=0)
        tot = (sums + dsum_ref[:, 0, :]) * inv_n
        logits = jnp.dot(tot, wcls_ref[...],
                         preferred_element_type=jnp.float32) + b_ref[...]
        m = jnp.max(logits, axis=-1, keepdims=True)
        e = jnp.exp(logits - m)
        out_ref[...] = e / jnp.sum(e, axis=-1, keepdims=True)

    return pl.pallas_call(
        body,
        grid=(B // BB,),
        in_specs=[
            pl.BlockSpec((2, BB * RPB, 128), lambda b: (0, b, 0)),
            pl.BlockSpec((BB, 1, G), lambda b: (b, 0, 0)),
            pl.BlockSpec((128, NPR * G), lambda b: (0, 0)),
            pl.BlockSpec((G, O), lambda b: (0, 0)),
            pl.BlockSpec((1, O), lambda b: (0, 0)),
        ],
        out_specs=pl.BlockSpec((BB, O), lambda b: (b, 0)),
        out_shape=jax.ShapeDtypeStruct((B, O), jnp.float32),
        compiler_params=pltpu.CompilerParams(
            dimension_semantics=("arbitrary",)),
    )(partials, dsum, wsim_diag, wcls, bcls)


def kernel(true_batch_size, node_list, edge_index, fw_adj_matrix,
           bw_adj_matrix, W_sim, W_fw, W_bw, W_cls, b_cls):
    B, N, F = node_list.shape
    dsum = _tc_dense(fw_adj_matrix, bw_adj_matrix, node_list, W_fw, W_bw)
    partials = _sc_scatter_partials(node_list, edge_index)
    wsim_diag = jnp.kron(jnp.eye(128 // F, dtype=W_sim.dtype), W_sim)
    return _tc_combine(partials, dsum, wsim_diag, W_cls,
                       b_cls.reshape(1, -1), N, F)


# speedup vs baseline: 54.7608x; 1.0647x over previous
import functools

import jax
import jax.numpy as jnp
from jax import lax
from jax.experimental import pallas as pl
from jax.experimental.pallas import tpu as pltpu
from jax.experimental.pallas import tpu_sc as plsc

_EDGE_CHUNK = 128
_NUM_SC = 2
_NUM_TILES = 16


def _sc_scatter_partials(node_list, edge_index):
    B, N, F = node_list.shape
    BN = B * N
    CH = _EDGE_CHUNK
    E = edge_index.shape[1]
    nchunks_total = E // CH
    NW = _NUM_SC * _NUM_TILES
    chunks_per_worker = nchunks_total // NW
    rows_per_tile = BN // _NUM_TILES

    mesh = plsc.VectorSubcoreMesh(core_axis_name="c", subcore_axis_name="s")

    GRP = 8
    qchunks = chunks_per_worker // 4
    ngroups = qchunks // GRP

    NPR = 128 // F
    out_rows = BN // NPR

    @functools.partial(
        pl.kernel,
        out_type=jax.ShapeDtypeStruct((_NUM_SC, out_rows, 128), jnp.float32),
        mesh=mesh,
        scratch_types=[
            pltpu.VMEM((qchunks * CH,), jnp.int32),
            pltpu.VMEM((qchunks, CH), jnp.int32),
            pltpu.VMEM((2 * GRP, CH, F), jnp.float32),
            pltpu.VMEM((CH, F), jnp.float32),
            pltpu.VMEM((CH, F), jnp.float32),
            pltpu.VMEM((CH // NPR, 128), jnp.float32),
            pltpu.VMEM_SHARED((BN, F), jnp.float32),
            pltpu.VMEM_SHARED((BN, F), jnp.float32),
            pltpu.SemaphoreType.DMA,
            pltpu.SemaphoreType.DMA,
        ],
        compiler_params=pltpu.CompilerParams(use_tc_tiling_on_sc=False),
    )
    def k(flat_hbm, eidx_hbm, out_hbm, src_v, dst_v, bufs, zbuf, wbuf, wbuf2,
          acc, table, gsem, ssem):
        c = lax.axis_index("c")
        s = lax.axis_index("s")
        wid = c * _NUM_TILES + s

        @pl.loop(0, CH)
        def _(i):
            zbuf[i, :] = jnp.zeros((F,), jnp.float32)

        row0 = s * rows_per_tile

        @pl.loop(0, rows_per_tile // CH)
        def _(i):
            pltpu.sync_copy(zbuf, acc.at[pl.ds(row0 + i * CH, CH)])

        bpt = rows_per_tile // N

        @pl.loop(0, bpt)
        def _(i):
            pltpu.sync_copy(flat_hbm.at[s * bpt + i],
                            table.at[pl.ds(row0 + i * N, N)])

        plsc.subcore_barrier()

        def gather(j, slot):
            pltpu.async_copy(
                table.at[src_v.at[pl.ds(j * CH, CH)]], bufs.at[slot], gsem)

        def gwait(j, slot):
            pltpu.make_async_copy(
                table.at[src_v.at[pl.ds(j * CH, CH)]], bufs.at[slot],
                gsem).wait()

        def scat(j, slot):
            pltpu.async_copy(bufs.at[slot], acc.at[dst_v.at[j]], ssem,
                             add=True)

        def swait(j, slot):
            pltpu.make_async_copy(
                bufs.at[slot], acc.at[dst_v.at[j]], ssem).wait()

        @pl.loop(0, 4)
        def _(h):
            ebase = (wid * chunks_per_worker + h * qchunks) * CH
            pltpu.async_copy(eidx_hbm.at[0, pl.ds(ebase, qchunks * CH)],
                             src_v, gsem)

            @pl.loop(0, qchunks)
            def _(r):
                pltpu.async_copy(eidx_hbm.at[1, pl.ds(ebase + r * CH, CH)],
                                 dst_v.at[r], gsem)

            pltpu.make_async_copy(
                eidx_hbm.at[0, pl.ds(ebase, qchunks * CH)], src_v,
                gsem).wait()

            @pl.loop(0, qchunks)
            def _(r):
                pltpu.make_async_copy(
                    eidx_hbm.at[1, pl.ds(ebase, CH)], dst_v.at[0],
                    gsem).wait()

            for b in range(GRP):
                gather(b, b)

            @pl.loop(0, ngroups, step=2)
            def _(g):
                ja = g * GRP
                jb = ja + GRP
                for b in range(GRP):
                    gwait(ja + b, b)
                @pl.when(g > 0)
                def _():
                    for b in range(GRP):
                        swait(jb - 2 * GRP + b, GRP + b)
                for b in range(GRP):
                    gather(jb + b, GRP + b)
                for b in range(GRP):
                    scat(ja + b, b)
                for b in range(GRP):
                    gwait(jb + b, GRP + b)
                for b in range(GRP):
                    swait(ja + b, b)

                @pl.when(g + 2 < ngroups)
                def _():
                    for b in range(GRP):
                        gather(jb + GRP + b, b)

                for b in range(GRP):
                    scat(jb + b, GRP + b)

            for b in range(GRP):
                swait((ngroups - 1) * GRP + b, GRP + b)

        plsc.subcore_barrier()

        @pl.loop(0, rows_per_tile // CH)
        def _(kblk):
            pltpu.sync_copy(acc.at[pl.ds(row0 + kblk * CH, CH)], wbuf)
            for j in range(CH):
                wbuf2[j // NPR, pl.ds((j % NPR) * F, F)] = wbuf[j, :]
            pltpu.sync_copy(
                wbuf2,
                out_hbm.at[c, pl.ds(s * (rows_per_tile // NPR)
                                    + kblk * (CH // NPR), CH // NPR)])

    return k(node_list, edge_index)


def _tc_dense(afw, abw, x, wf, wb):
    B, N, F = x.shape
    G = wf.shape[-1]

    BB = 2

    def body(afw_ref, abw_ref, x_ref, wf_ref, wb_ref, out_ref):
        for i in range(BB):
            xb = x_ref[i]
            xwf = jnp.dot(
                xb, wf_ref[...],
                preferred_element_type=jnp.float32).astype(jnp.bfloat16)
            xwb = jnp.dot(
                xb, wb_ref[...],
                preferred_element_type=jnp.float32).astype(jnp.bfloat16)
            hf = jnp.maximum(
                jnp.dot(afw_ref[i].astype(jnp.bfloat16), xwf,
                        preferred_element_type=jnp.float32), 0.0)
            hb = jnp.maximum(
                jnp.dot(abw_ref[i].astype(jnp.bfloat16), xwb,
                        preferred_element_type=jnp.float32), 0.0)
            out_ref[i, 0, :] = jnp.sum(hf + hb, axis=0)

    return pl.pallas_call(
        body,
        grid=(B // BB,),
        in_specs=[
            pl.BlockSpec((BB, N, N), lambda b: (b, 0, 0)),
            pl.BlockSpec((BB, N, N), lambda b: (b, 0, 0)),
            pl.BlockSpec((BB, N, F), lambda b: (b, 0, 0)),
            pl.BlockSpec((F, G), lambda b: (0, 0)),
            pl.BlockSpec((F, G), lambda b: (0, 0)),
        ],
        out_specs=pl.BlockSpec((BB, 1, G), lambda b: (b, 0, 0)),
        out_shape=jax.ShapeDtypeStruct((B, 1, G), jnp.float32),
        compiler_params=pltpu.CompilerParams(
            dimension_semantics=("arbitrary",)),
    )(afw, abw, x, wf, wb)


def _tc_combine(partials, dsum, wsim_diag, wcls, bcls, n_nodes, n_feat):
    _, out_rows, _ = partials.shape
    N = n_nodes
    F = n_feat
    NPR = 128 // F
    G = wsim_diag.shape[-1] // NPR
    B = out_rows * NPR // N
    O = wcls.shape[-1]
    inv_n = 1.0 / N
    BB = 8
    RPB = N // NPR

    def body(p_ref, dsum_ref, wsim_ref, wcls_ref, b_ref, out_ref):
        agg = p_ref[0] + p_ref[1]
        sim = jnp.maximum(
            jnp.dot(agg, wsim_ref[...], preferred_element_type=jnp.float32),
            0.0)
        rows = []
        for b in range(BB):
            v = jnp.sum(sim[b * RPB:(b + 1) * RPB], axis=0)
            acc = v[0:G]
            for j in range(1, NPR):
                acc = acc + v[j * G:(j + 1) * G]
            rows.append(acc.reshape(1, G))
        sums = jnp.concatenate(rows, axis=0)
        tot = (sums + dsum_ref[:, 0, :]) * inv_n
        logits = jnp.dot(tot, wcls_ref[...],
                         preferred_element_type=jnp.float32) + b_ref[...]
        m = jnp.max(logits, axis=-1, keepdims=True)
        e = jnp.exp(logits - m)
        out_ref[...] = e / jnp.sum(e, axis=-1, keepdims=True)

    return pl.pallas_call(
        body,
        grid=(B // BB,),
        in_specs=[
            pl.BlockSpec((2, BB * RPB, 128), lambda b: (0, b, 0)),
            pl.BlockSpec((BB, 1, G), lambda b: (b, 0, 0)),
            pl.BlockSpec((128, NPR * G), lambda b: (0, 0)),
            pl.BlockSpec((G, O), lambda b: (0, 0)),
            pl.BlockSpec((1, O), lambda b: (0, 0)),
        ],
        out_specs=pl.BlockSpec((BB, O), lambda b: (b, 0)),
        out_shape=jax.ShapeDtypeStruct((B, O), jnp.float32),
        compiler_params=pltpu.CompilerParams(
            dimension_semantics=("arbitrary",)),
    )(partials, dsum, wsim_diag, wcls, bcls)


def kernel(true_batch_size, node_list, edge_index, fw_adj_matrix,
           bw_adj_matrix, W_sim, W_fw, W_bw, W_cls, b_cls):
    B, N, F = node_list.shape
    dsum = _tc_dense(fw_adj_matrix, bw_adj_matrix, node_list, W_fw, W_bw)
    partials = _sc_scatter_partials(node_list, edge_index)
    wsim_diag = jnp.kron(jnp.eye(128 // F, dtype=W_sim.dtype), W_sim)
    return _tc_combine(partials, dsum, wsim_diag, W_cls,
                       b_cls.reshape(1, -1), N, F)


# speedup vs baseline: 57.5215x; 1.0504x over previous
import functools

import jax
import jax.numpy as jnp
from jax import lax
from jax.experimental import pallas as pl
from jax.experimental.pallas import tpu as pltpu
from jax.experimental.pallas import tpu_sc as plsc

_EDGE_CHUNK = 128
_NUM_SC = 2
_NUM_TILES = 16


def _sc_scatter_partials(node_list, eidx_il):
    B, N, F = node_list.shape
    BN = B * N
    CH = _EDGE_CHUNK
    nchunks_total = eidx_il.shape[0] // 2
    NW = _NUM_SC * _NUM_TILES
    chunks_per_worker = nchunks_total // NW
    rows_per_tile = BN // _NUM_TILES

    mesh = plsc.VectorSubcoreMesh(core_axis_name="c", subcore_axis_name="s")

    GRP = 8
    qchunks = chunks_per_worker // 4
    ngroups = qchunks // GRP

    NPR = 128 // F
    out_rows = BN // NPR

    @functools.partial(
        pl.kernel,
        out_type=jax.ShapeDtypeStruct((_NUM_SC, out_rows, 128), jnp.float32),
        mesh=mesh,
        scratch_types=[
            pltpu.VMEM((2 * qchunks, CH), jnp.int32),
            pltpu.VMEM((2 * GRP, CH, F), jnp.float32),
            pltpu.VMEM((CH, F), jnp.float32),
            pltpu.VMEM((CH, F), jnp.float32),
            pltpu.VMEM((CH // NPR, 128), jnp.float32),
            pltpu.VMEM_SHARED((BN, F), jnp.float32),
            pltpu.VMEM_SHARED((BN, F), jnp.float32),
            pltpu.SemaphoreType.DMA,
            pltpu.SemaphoreType.DMA,
        ],
        compiler_params=pltpu.CompilerParams(use_tc_tiling_on_sc=False),
    )
    def k(flat_hbm, eidx_hbm, out_hbm, idx_v, bufs, zbuf, wbuf, wbuf2,
          acc, table, gsem, ssem):
        c = lax.axis_index("c")
        s = lax.axis_index("s")
        wid = c * _NUM_TILES + s

        @pl.loop(0, CH)
        def _(i):
            zbuf[i, :] = jnp.zeros((F,), jnp.float32)

        row0 = s * rows_per_tile

        @pl.loop(0, rows_per_tile // CH)
        def _(i):
            pltpu.sync_copy(zbuf, acc.at[pl.ds(row0 + i * CH, CH)])

        bpt = rows_per_tile // N

        @pl.loop(0, bpt)
        def _(i):
            pltpu.sync_copy(flat_hbm.at[s * bpt + i],
                            table.at[pl.ds(row0 + i * N, N)])

        plsc.subcore_barrier()

        def gather(j, slot):
            pltpu.async_copy(table.at[idx_v.at[2 * j]], bufs.at[slot], gsem)

        def gwait(j, slot):
            pltpu.make_async_copy(
                table.at[idx_v.at[2 * j]], bufs.at[slot], gsem).wait()

        def scat(j, slot):
            pltpu.async_copy(bufs.at[slot], acc.at[idx_v.at[2 * j + 1]],
                             ssem, add=True)

        def swait(j, slot):
            pltpu.make_async_copy(
                bufs.at[slot], acc.at[idx_v.at[2 * j + 1]], ssem).wait()

        @pl.loop(0, 4)
        def _(h):
            cbase = 2 * (wid * chunks_per_worker + h * qchunks)
            pltpu.sync_copy(eidx_hbm.at[pl.ds(cbase, 2 * qchunks)], idx_v)

            for b in range(GRP):
                gather(b, b)

            @pl.loop(0, ngroups, step=2)
            def _(g):
                ja = g * GRP
                jb = ja + GRP
                for b in range(GRP):
                    gwait(ja + b, b)
                @pl.when(g > 0)
                def _():
                    for b in range(GRP):
                        swait(jb - 2 * GRP + b, GRP + b)
                for b in range(GRP):
                    gather(jb + b, GRP + b)
                for b in range(GRP):
                    scat(ja + b, b)
                for b in range(GRP):
                    gwait(jb + b, GRP + b)
                for b in range(GRP):
                    swait(ja + b, b)

                @pl.when(g + 2 < ngroups)
                def _():
                    for b in range(GRP):
                        gather(jb + GRP + b, b)

                for b in range(GRP):
                    scat(jb + b, GRP + b)

            for b in range(GRP):
                swait((ngroups - 1) * GRP + b, GRP + b)

        plsc.subcore_barrier()

        @pl.loop(0, rows_per_tile // CH)
        def _(kblk):
            pltpu.sync_copy(acc.at[pl.ds(row0 + kblk * CH, CH)], wbuf)
            for j in range(CH):
                wbuf2[j // NPR, pl.ds((j % NPR) * F, F)] = wbuf[j, :]
            pltpu.sync_copy(
                wbuf2,
                out_hbm.at[c, pl.ds(s * (rows_per_tile // NPR)
                                    + kblk * (CH // NPR), CH // NPR)])

    return k(node_list, eidx_il)


def _tc_dense(afw, abw, x, wf, wb):
    B, N, F = x.shape
    G = wf.shape[-1]

    BB = 2

    def body(afw_ref, abw_ref, x_ref, wf_ref, wb_ref, out_ref):
        for i in range(BB):
            xb = x_ref[i]
            xwf = jnp.dot(
                xb, wf_ref[...],
                preferred_element_type=jnp.float32).astype(jnp.bfloat16)
            xwb = jnp.dot(
                xb, wb_ref[...],
                preferred_element_type=jnp.float32).astype(jnp.bfloat16)
            hf = jnp.maximum(
                jnp.dot(afw_ref[i].astype(jnp.bfloat16), xwf,
                        preferred_element_type=jnp.float32), 0.0)
            hb = jnp.maximum(
                jnp.dot(abw_ref[i].astype(jnp.bfloat16), xwb,
                        preferred_element_type=jnp.float32), 0.0)
            out_ref[i, 0, :] = jnp.sum(hf + hb, axis=0)

    return pl.pallas_call(
        body,
        grid=(B // BB,),
        in_specs=[
            pl.BlockSpec((BB, N, N), lambda b: (b, 0, 0)),
            pl.BlockSpec((BB, N, N), lambda b: (b, 0, 0)),
            pl.BlockSpec((BB, N, F), lambda b: (b, 0, 0)),
            pl.BlockSpec((F, G), lambda b: (0, 0)),
            pl.BlockSpec((F, G), lambda b: (0, 0)),
        ],
        out_specs=pl.BlockSpec((BB, 1, G), lambda b: (b, 0, 0)),
        out_shape=jax.ShapeDtypeStruct((B, 1, G), jnp.float32),
        compiler_params=pltpu.CompilerParams(
            dimension_semantics=("arbitrary",)),
    )(afw, abw, x, wf, wb)


def _tc_combine(partials, dsum, wsim_diag, wcls, bcls, n_nodes, n_feat):
    _, out_rows, _ = partials.shape
    N = n_nodes
    F = n_feat
    NPR = 128 // F
    G = wsim_diag.shape[-1] // NPR
    B = out_rows * NPR // N
    O = wcls.shape[-1]
    inv_n = 1.0 / N
    BB = 8
    RPB = N // NPR

    def body(p_ref, dsum_ref, wsim_ref, wcls_ref, b_ref, out_ref):
        agg = p_ref[0] + p_ref[1]
        sim = jnp.maximum(
            jnp.dot(agg, wsim_ref[...], preferred_element_type=jnp.float32),
            0.0)
        rows = []
        for b in range(BB):
            v = jnp.sum(sim[b * RPB:(b + 1) * RPB], axis=0)
            acc = v[0:G]
            for j in range(1, NPR):
                acc = acc + v[j * G:(j + 1) * G]
            rows.append(acc.reshape(1, G))
        sums = jnp.concatenate(rows, axis=0)
        tot = (sums + dsum_ref[:, 0, :]) * inv_n
        logits = jnp.dot(tot, wcls_ref[...],
                         preferred_element_type=jnp.float32) + b_ref[...]
        m = jnp.max(logits, axis=-1, keepdims=True)
        e = jnp.exp(logits - m)
        out_ref[...] = e / jnp.sum(e, axis=-1, keepdims=True)

    return pl.pallas_call(
        body,
        grid=(B // BB,),
        in_specs=[
            pl.BlockSpec((2, BB * RPB, 128), lambda b: (0, b, 0)),
            pl.BlockSpec((BB, 1, G), lambda b: (b, 0, 0)),
            pl.BlockSpec((128, NPR * G), lambda b: (0, 0)),
            pl.BlockSpec((G, O), lambda b: (0, 0)),
            pl.BlockSpec((1, O), lambda b: (0, 0)),
        ],
        out_specs=pl.BlockSpec((BB, O), lambda b: (b, 0)),
        out_shape=jax.ShapeDtypeStruct((B, O), jnp.float32),
        compiler_params=pltpu.CompilerParams(
            dimension_semantics=("arbitrary",)),
    )(partials, dsum, wsim_diag, wcls, bcls)


def kernel(true_batch_size, node_list, edge_index, fw_adj_matrix,
           bw_adj_matrix, W_sim, W_fw, W_bw, W_cls, b_cls):
    B, N, F = node_list.shape
    dsum = _tc_dense(fw_adj_matrix, bw_adj_matrix, node_list, W_fw, W_bw)
    E = edge_index.shape[1]
    nchunks = E // _EDGE_CHUNK
    eidx_il = (edge_index.reshape(2, nchunks, _EDGE_CHUNK)
               .transpose(1, 0, 2).reshape(2 * nchunks, _EDGE_CHUNK))
    partials = _sc_scatter_partials(node_list, eidx_il)
    wsim_diag = jnp.kron(jnp.eye(128 // F, dtype=W_sim.dtype), W_sim)
    return _tc_combine(partials, dsum, wsim_diag, W_cls,
                       b_cls.reshape(1, -1), N, F)


# speedup vs baseline: 61.0974x; 1.0622x over previous
import dataclasses
import functools

import jax
import jax.numpy as jnp
from jax import lax
from jax.experimental import pallas as pl
from jax.experimental.pallas import tpu as pltpu
from jax.experimental.pallas import tpu_sc as plsc

_EDGE_CHUNK = 128
_NUM_SC = 2
_NUM_TILES = 16


def _sc_compiler_params():
    cp = pltpu.CompilerParams(use_tc_tiling_on_sc=False)
    if "needs_layout_passes" in pltpu.CompilerParams.__dataclass_fields__:
        cp = dataclasses.replace(cp, needs_layout_passes=False)
    return cp


def _sc_scatter_partials(nlT, eidx_il):
    B, F, N = nlT.shape
    BN = B * N
    CH = _EDGE_CHUNK
    nchunks_total = eidx_il.shape[0] // 2
    NW = _NUM_SC * _NUM_TILES
    chunks_per_worker = nchunks_total // NW
    rows_per_tile = BN // _NUM_TILES

    mesh = plsc.VectorSubcoreMesh(core_axis_name="c", subcore_axis_name="s")

    GRP = 8
    qchunks = chunks_per_worker // 4
    ngroups = qchunks // GRP

    NPR = 128 // F
    out_rows = BN // NPR

    @functools.partial(
        pl.kernel,
        out_type=jax.ShapeDtypeStruct((_NUM_SC, out_rows, 128), jnp.float32),
        mesh=mesh,
        scratch_types=[
            pltpu.VMEM((2 * qchunks, CH), jnp.int32),
            pltpu.VMEM((2 * GRP, CH, F), jnp.float32),
            pltpu.VMEM((CH, F), jnp.float32),
            pltpu.VMEM((CH, F), jnp.float32),
            pltpu.VMEM((CH // NPR, 128), jnp.float32),
            pltpu.VMEM((F, N), jnp.float32),
            pltpu.VMEM_SHARED((BN, F), jnp.float32),
            pltpu.VMEM_SHARED((BN, F), jnp.float32),
            pltpu.SemaphoreType.DMA,
            pltpu.SemaphoreType.DMA,
        ],
        compiler_params=_sc_compiler_params(),
    )
    def k(flat_hbm, eidx_hbm, out_hbm, idx_v, bufs, zbuf, wbuf, wbuf2, fbuf,
          acc, table, gsem, ssem):
        c = lax.axis_index("c")
        s = lax.axis_index("s")
        wid = c * _NUM_TILES + s

        @pl.loop(0, CH)
        def _(i):
            zbuf[i, :] = jnp.zeros((F,), jnp.float32)

        row0 = s * rows_per_tile

        @pl.loop(0, rows_per_tile // CH)
        def _(i):
            pltpu.sync_copy(zbuf, acc.at[pl.ds(row0 + i * CH, CH)])

        bpt = rows_per_tile // N
        lane = lax.iota(jnp.int32, 16)

        @pl.loop(0, bpt)
        def _(i):
            pltpu.sync_copy(flat_hbm.at[s * bpt + i], fbuf)

            @pl.loop(0, N // CH)
            def _(nc):
                for g in range(CH // 16):
                    for f in range(F):
                        v = fbuf[f, pl.ds(nc * CH + g * 16, 16)]
                        plsc.store_scatter(
                            wbuf, [g * 16 + lane,
                                   jnp.full((16,), f, jnp.int32)], v)
                pltpu.sync_copy(
                    wbuf, table.at[pl.ds(row0 + i * N + nc * CH, CH)])

        plsc.subcore_barrier()

        def gather(j, slot):
            pltpu.async_copy(table.at[idx_v.at[2 * j]], bufs.at[slot], gsem)

        def gwait(j, slot):
            pltpu.make_async_copy(
                table.at[idx_v.at[2 * j]], bufs.at[slot], gsem).wait()

        def scat(j, slot):
            pltpu.async_copy(bufs.at[slot], acc.at[idx_v.at[2 * j + 1]],
                             ssem, add=True)

        def swait(j, slot):
            pltpu.make_async_copy(
                bufs.at[slot], acc.at[idx_v.at[2 * j + 1]], ssem).wait()

        @pl.loop(0, 4)
        def _(h):
            cbase = 2 * (wid * chunks_per_worker + h * qchunks)
            pltpu.sync_copy(eidx_hbm.at[pl.ds(cbase, 2 * qchunks)], idx_v)

            for b in range(GRP):
                gather(b, b)

            @pl.loop(0, ngroups, step=2)
            def _(g):
                ja = g * GRP
                jb = ja + GRP
                for b in range(GRP):
                    gwait(ja + b, b)
                @pl.when(g > 0)
                def _():
                    for b in range(GRP):
                        swait(jb - 2 * GRP + b, GRP + b)
                for b in range(GRP):
                    gather(jb + b, GRP + b)
                for b in range(GRP):
                    scat(ja + b, b)
                for b in range(GRP):
                    gwait(jb + b, GRP + b)
                for b in range(GRP):
                    swait(ja + b, b)

                @pl.when(g + 2 < ngroups)
                def _():
                    for b in range(GRP):
                        gather(jb + GRP + b, b)

                for b in range(GRP):
                    scat(jb + b, GRP + b)

            for b in range(GRP):
                swait((ngroups - 1) * GRP + b, GRP + b)

        plsc.subcore_barrier()

        @pl.loop(0, rows_per_tile // CH)
        def _(kblk):
            pltpu.sync_copy(acc.at[pl.ds(row0 + kblk * CH, CH)], wbuf)
            for j in range(CH):
                wbuf2[j // NPR, pl.ds((j % NPR) * F, F)] = wbuf[j, :]
            pltpu.sync_copy(
                wbuf2,
                out_hbm.at[c, pl.ds(s * (rows_per_tile // NPR)
                                    + kblk * (CH // NPR), CH // NPR)])

    return k(nlT, eidx_il)


def _tc_dense(afw, abw, x, wf, wb):
    B, N, F = x.shape
    G = wf.shape[-1]

    BB = 2

    def body(afw_ref, abw_ref, x_ref, wf_ref, wb_ref, out_ref):
        for i in range(BB):
            xb = x_ref[i]
            xwf = jnp.dot(
                xb, wf_ref[...],
                preferred_element_type=jnp.float32).astype(jnp.bfloat16)
            xwb = jnp.dot(
                xb, wb_ref[...],
                preferred_element_type=jnp.float32).astype(jnp.bfloat16)
            hf = jnp.maximum(
                jnp.dot(afw_ref[i].astype(jnp.bfloat16), xwf,
                        preferred_element_type=jnp.float32), 0.0)
            hb = jnp.maximum(
                jnp.dot(abw_ref[i].astype(jnp.bfloat16), xwb,
                        preferred_element_type=jnp.float32), 0.0)
            out_ref[i, 0, :] = jnp.sum(hf + hb, axis=0)

    return pl.pallas_call(
        body,
        grid=(B // BB,),
        in_specs=[
            pl.BlockSpec((BB, N, N), lambda b: (b, 0, 0)),
            pl.BlockSpec((BB, N, N), lambda b: (b, 0, 0)),
            pl.BlockSpec((BB, N, F), lambda b: (b, 0, 0)),
            pl.BlockSpec((F, G), lambda b: (0, 0)),
            pl.BlockSpec((F, G), lambda b: (0, 0)),
        ],
        out_specs=pl.BlockSpec((BB, 1, G), lambda b: (b, 0, 0)),
        out_shape=jax.ShapeDtypeStruct((B, 1, G), jnp.float32),
        compiler_params=pltpu.CompilerParams(
            dimension_semantics=("arbitrary",)),
    )(afw, abw, x, wf, wb)


def _tc_combine(partials, dsum, wsim_diag, wcls, bcls, n_nodes, n_feat):
    _, out_rows, _ = partials.shape
    N = n_nodes
    F = n_feat
    NPR = 128 // F
    G = wsim_diag.shape[-1] // NPR
    B = out_rows * NPR // N
    O = wcls.shape[-1]
    inv_n = 1.0 / N
    BB = 8
    RPB = N // NPR

    def body(p_ref, dsum_ref, wsim_ref, wcls_ref, b_ref, out_ref):
        agg = p_ref[0] + p_ref[1]
        sim = jnp.maximum(
            jnp.dot(agg, wsim_ref[...], preferred_element_type=jnp.float32),
            0.0)
        rows = []
        for b in range(BB):
            v = jnp.sum(sim[b * RPB:(b + 1) * RPB], axis=0)
            acc = v[0:G]
            for j in range(1, NPR):
                acc = acc + v[j * G:(j + 1) * G]
            rows.append(acc.reshape(1, G))
        sums = jnp.concatenate(rows, axis=0)
        tot = (sums + dsum_ref[:, 0, :]) * inv_n
        logits = jnp.dot(tot, wcls_ref[...],
                         preferred_element_type=jnp.float32) + b_ref[...]
        m = jnp.max(logits, axis=-1, keepdims=True)
        e = jnp.exp(logits - m)
        out_ref[...] = e / jnp.sum(e, axis=-1, keepdims=True)

    return pl.pallas_call(
        body,
        grid=(B // BB,),
        in_specs=[
            pl.BlockSpec((2, BB * RPB, 128), lambda b: (0, b, 0)),
            pl.BlockSpec((BB, 1, G), lambda b: (b, 0, 0)),
            pl.BlockSpec((128, NPR * G), lambda b: (0, 0)),
            pl.BlockSpec((G, O), lambda b: (0, 0)),
            pl.BlockSpec((1, O), lambda b: (0, 0)),
        ],
        out_specs=pl.BlockSpec((BB, O), lambda b: (b, 0)),
        out_shape=jax.ShapeDtypeStruct((B, O), jnp.float32),
        compiler_params=pltpu.CompilerParams(
            dimension_semantics=("arbitrary",)),
    )(partials, dsum, wsim_diag, wcls, bcls)


def kernel(true_batch_size, node_list, edge_index, fw_adj_matrix,
           bw_adj_matrix, W_sim, W_fw, W_bw, W_cls, b_cls):
    B, N, F = node_list.shape
    dsum = _tc_dense(fw_adj_matrix, bw_adj_matrix, node_list, W_fw, W_bw)
    E = edge_index.shape[1]
    nchunks = E // _EDGE_CHUNK
    eidx_il = (edge_index.reshape(2, nchunks, _EDGE_CHUNK)
               .transpose(1, 0, 2).reshape(2 * nchunks, _EDGE_CHUNK))
    nlT = jnp.transpose(node_list, (0, 2, 1))
    partials = _sc_scatter_partials(nlT, eidx_il)
    wsim_diag = jnp.kron(jnp.eye(128 // F, dtype=W_sim.dtype), W_sim)
    return _tc_combine(partials, dsum, wsim_diag, W_cls,
                       b_cls.reshape(1, -1), N, F)
